# baseline (speedup 1.0000x reference)
import functools

import jax
import jax.numpy as jnp
import numpy as _np
from jax import lax
from jax.experimental import pallas as pl
from jax.experimental.pallas import tpu as pltpu
from jax.experimental.pallas import tpu_sc as plsc

N = 10000
E = 320000
E_LG = 640000
F = 128
TWO_F = 256
NW = 32
T = 128

_INTERPRET = False
_I0 = _np.int32(0)

N_PAD = 10240
NTW_G = 79
EP_G = NW * NTW_G * T
ROWS_PER_SUB = N_PAD // 16
ZB = 64


def _make_segsum_g():
    mesh = plsc.VectorSubcoreMesh(core_axis_name="c", subcore_axis_name="s")

    @functools.partial(
        pl.kernel,
        mesh=mesh,
        out_type=jax.ShapeDtypeStruct((2 * N_PAD, F), jnp.float32),
        scratch_types=[
            pltpu.VMEM((T,), jnp.int32),
            pltpu.VMEM((T,), jnp.int32),
            pltpu.VMEM((T, F), jnp.float32),
            pltpu.VMEM((ZB, F), jnp.float32),
            pltpu.VMEM_SHARED((N_PAD, F), jnp.float32),
            pltpu.SemaphoreType.DMA,
        ],
    )
    def seg_kernel(vals_hbm, src_hbm, dst_hbm, out_hbm,
                   sidx, sdst, grows, zbuf, accs, sem):
        sc = lax.axis_index("c")
        sub = lax.axis_index("s")
        w = sc * 16 + sub
        zero16f = jnp.zeros((16,), jnp.float32)
        for r in range(ZB):
            for k in range(F // 16):
                zbuf[r, pl.ds(k * 16, 16)] = zero16f
        row0 = sub * ROWS_PER_SUB
        for j in range(ROWS_PER_SUB // ZB):
            pltpu.async_copy(zbuf, accs.at[pl.ds(row0 + j * ZB, ZB)],
                             sem).wait()
        plsc.subcore_barrier()

        base = w * (NTW_G * T)
        for j in range(NTW_G):
            pos = base + j * T
            pltpu.async_copy(src_hbm.at[pl.ds(pos, T)], sidx, sem).wait()
            pltpu.async_copy(dst_hbm.at[pl.ds(pos, T)], sdst, sem).wait()
            pltpu.async_copy(vals_hbm.at[sidx], grows, sem).wait()
            pltpu.async_copy(grows, accs.at[sdst], sem, add=True).wait()
        plsc.subcore_barrier()
        pltpu.async_copy(
            accs.at[pl.ds(row0, ROWS_PER_SUB)],
            out_hbm.at[pl.ds(sc * N_PAD + row0, ROWS_PER_SUB)], sem).wait()

    return seg_kernel


_SEG_G = _make_segsum_g()


def _segsum_g(vals, src_pad, dst_pad):
    return _SEG_G(vals, src_pad, dst_pad)


def _add_body(a_ref, b_ref, o_ref):
    o_ref[...] = a_ref[...] + b_ref[...]


def _combine(p, block=1024):
    return pl.pallas_call(
        _add_body,
        grid=(N_PAD // block,),
        in_specs=[
            pl.BlockSpec((block, F), lambda i: (i, _I0)),
            pl.BlockSpec((block, F), lambda i: (i + N_PAD // block, _I0)),
        ],
        out_specs=pl.BlockSpec((block, F), lambda i: (i, _I0)),
        out_shape=jax.ShapeDtypeStruct((N_PAD, F), jnp.float32),
        interpret=_INTERPRET,
    )(p, p)


def _dense_body(z_ref, z1_ref, z2_ref, agg_ref, deg_ref,
                w_ref, b_ref, t_ref, s1_ref, s2_ref):
    z = z_ref[...]
    xn = (jnp.dot(z, w_ref[0], preferred_element_type=jnp.float32)
          + deg_ref[...] * jnp.dot(z, w_ref[1], preferred_element_type=jnp.float32)
          + jnp.dot(z1_ref[...], w_ref[2], preferred_element_type=jnp.float32)
          + jnp.dot(z2_ref[...], w_ref[3], preferred_element_type=jnp.float32)
          + jnp.dot(agg_ref[...], w_ref[4], preferred_element_type=jnp.float32)
          + b_ref[...])
    t = xn[:, :F] + jax.nn.relu(xn[:, F:])
    t_ref[...] = t
    s1_ref[...] = jnp.sum(t, axis=0, keepdims=True)[None]
    s2_ref[...] = jnp.sum(t * t, axis=0, keepdims=True)[None]


def _dense_stage(z, z1, z2, agg, deg, w_stack, b_sum, block):
    m = z.shape[0]
    grid = m // block
    t, s1, s2 = pl.pallas_call(
        _dense_body,
        grid=(grid,),
        in_specs=[
            pl.BlockSpec((block, F), lambda i: (i, _I0)),
            pl.BlockSpec((block, F), lambda i: (i, _I0)),
            pl.BlockSpec((block, F), lambda i: (i, _I0)),
            pl.BlockSpec((block, F), lambda i: (i, _I0)),
            pl.BlockSpec((block, 1), lambda i: (i, _I0)),
            pl.BlockSpec((5, F, TWO_F), lambda i: (_I0, _I0, _I0)),
            pl.BlockSpec((1, TWO_F), lambda i: (_I0, _I0)),
        ],
        out_specs=[
            pl.BlockSpec((block, F), lambda i: (i, _I0)),
            pl.BlockSpec((1, 1, F), lambda i: (i, _I0, _I0)),
            pl.BlockSpec((1, 1, F), lambda i: (i, _I0, _I0)),
        ],
        out_shape=[
            jax.ShapeDtypeStruct((m, F), jnp.float32),
            jax.ShapeDtypeStruct((grid, 1, F), jnp.float32),
            jax.ShapeDtypeStruct((grid, 1, F), jnp.float32),
        ],
        interpret=_INTERPRET,
    )(z, z1, z2, agg, deg, w_stack, b_sum)
    return t, s1, s2


def _norm_body(t_ref, a_ref, c_ref, o_ref):
    o_ref[...] = t_ref[...] * a_ref[...] + c_ref[...]


def _norm_stage(t, a, c, block):
    m = t.shape[0]
    return pl.pallas_call(
        _norm_body,
        grid=(m // block,),
        in_specs=[
            pl.BlockSpec((block, F), lambda i: (i, _I0)),
            pl.BlockSpec((1, F), lambda i: (_I0, _I0)),
            pl.BlockSpec((1, F), lambda i: (_I0, _I0)),
        ],
        out_specs=pl.BlockSpec((block, F), lambda i: (i, _I0)),
        out_shape=jax.ShapeDtypeStruct((m, F), jnp.float32),
        interpret=_INTERPRET,
    )(t, a, c)


def _branch(z, z1, z2, agg, deg, names, params, bn_w, bn_b, block):
    w_stack = jnp.stack([params[n][0] for n in names])
    b_sum = sum(params[n][1] for n in names).reshape(1, TWO_F).astype(jnp.float32)
    t, s1, s2 = _dense_stage(z, z1, z2, agg, deg, w_stack, b_sum, block)
    m = jnp.float32(z.shape[0])
    mean = jnp.sum(s1, axis=(0, 1)) / m
    var = jnp.sum(s2, axis=(0, 1)) / m - mean * mean
    rstd = lax.rsqrt(var + 1e-5)
    a = (rstd * bn_w).reshape(1, F)
    c = (bn_b - mean * rstd * bn_w).reshape(1, F)
    return _norm_stage(t, a, c, block)


def _pad_edges_g(src, dst):
    npad = EP_G - E
    pad_src = (jnp.arange(npad, dtype=jnp.int32) % N)
    pad_dst = N + (jnp.arange(npad, dtype=jnp.int32) % (N_PAD - N))
    return (jnp.concatenate([src, pad_src]), jnp.concatenate([dst, pad_dst]))


def kernel(x, y, deg_g, deg_lg, params, edge_index, edge_index_lg, eid2nid):
    src_g = edge_index[0].astype(jnp.int32)
    dst_g = edge_index[1].astype(jnp.int32)
    src_l = edge_index_lg[0].astype(jnp.int32)
    dst_l = edge_index_lg[1].astype(jnp.int32)
    nid = eid2nid.astype(jnp.int32)

    src_gp, dst_gp = _pad_edges_g(src_g, dst_g)
    pos_p, _ = _pad_edges_g(jnp.arange(E, dtype=jnp.int32), dst_g)

    z1g_p = _segsum_g(x, src_gp, dst_gp)
    z1g = _combine(z1g_p)
    z2g_p = _segsum_g(z1g, src_gp, dst_gp)
    z2g = _combine(z2g_p)
    yx_p = _segsum_g(y, pos_p, dst_gp)
    yx = _combine(yx_p)
    x_out = _branch(x, z1g, z2g, yx, deg_g,
                    ["theta_x", "theta_deg", "theta_0", "theta_1", "theta_y"],
                    params, params["bn_x_w"], params["bn_x_b"], 1000)

    z1l = jax.ops.segment_sum(y[src_l], dst_l, num_segments=E)
    z2l = jax.ops.segment_sum(z1l[src_l], dst_l, num_segments=E)
    nid_comp = jnp.take(nid, src_l, mode="clip")
    xy_agg = jax.ops.segment_sum(jnp.take(x, nid_comp, axis=0, mode="clip"),
                                 dst_l, num_segments=E)
    y_out = _branch(y, z1l, z2l, xy_agg, deg_lg,
                    ["gamma_y", "gamma_deg", "gamma_0", "gamma_1", "gamma_x"],
                    params, params["bn_y_w"], params["bn_y_b"], 1000)
    return (x_out, y_out)


# speedup vs baseline: 1.6421x; 1.6421x over previous
import functools

import jax
import jax.numpy as jnp
import numpy as _np
from jax import lax
from jax.experimental import pallas as pl
from jax.experimental.pallas import tpu as pltpu
from jax.experimental.pallas import tpu_sc as plsc

N = 10000
E = 320000
E_LG = 640000
F = 128
TWO_F = 256
NW = 32
T = 128

_INTERPRET = False
_I0 = _np.int32(0)

N_PAD = 10240
NTW_G = 79
EP_G = NW * NTW_G * T
ROWS_PER_SUB = N_PAD // 16
ZB = 64


def _make_segsum_g():
    mesh = plsc.VectorSubcoreMesh(core_axis_name="c", subcore_axis_name="s")

    @functools.partial(
        pl.kernel,
        mesh=mesh,
        out_type=jax.ShapeDtypeStruct((2 * N_PAD, F), jnp.float32),
        scratch_types=[
            pltpu.VMEM((T,), jnp.int32),
            pltpu.VMEM((T,), jnp.int32),
            pltpu.VMEM((T, F), jnp.float32),
            pltpu.VMEM((ZB, F), jnp.float32),
            pltpu.VMEM_SHARED((N_PAD, F), jnp.float32),
            pltpu.SemaphoreType.DMA,
        ],
    )
    def seg_kernel(vals_hbm, src_hbm, dst_hbm, out_hbm,
                   sidx, sdst, grows, zbuf, accs, sem):
        sc = lax.axis_index("c")
        sub = lax.axis_index("s")
        w = sc * 16 + sub
        zero16f = jnp.zeros((16,), jnp.float32)
        for r in range(ZB):
            for k in range(F // 16):
                zbuf[r, pl.ds(k * 16, 16)] = zero16f
        row0 = sub * ROWS_PER_SUB
        for j in range(ROWS_PER_SUB // ZB):
            pltpu.async_copy(zbuf, accs.at[pl.ds(row0 + j * ZB, ZB)],
                             sem).wait()
        plsc.subcore_barrier()

        base = w * (NTW_G * T)
        for j in range(NTW_G):
            pos = base + j * T
            pltpu.async_copy(src_hbm.at[pl.ds(pos, T)], sidx, sem).wait()
            pltpu.async_copy(dst_hbm.at[pl.ds(pos, T)], sdst, sem).wait()
            pltpu.async_copy(vals_hbm.at[sidx], grows, sem).wait()
            pltpu.async_copy(grows, accs.at[sdst], sem, add=True).wait()
        plsc.subcore_barrier()
        pltpu.async_copy(
            accs.at[pl.ds(row0, ROWS_PER_SUB)],
            out_hbm.at[pl.ds(sc * N_PAD + row0, ROWS_PER_SUB)], sem).wait()

    return seg_kernel


_SEG_G = _make_segsum_g()


def _segsum_g(vals, src_pad, dst_pad):
    return _SEG_G(vals, src_pad, dst_pad)


def _add_body(a_ref, b_ref, o_ref):
    o_ref[...] = a_ref[...] + b_ref[...]


def _combine(p, block=1024):
    return pl.pallas_call(
        _add_body,
        grid=(N_PAD // block,),
        in_specs=[
            pl.BlockSpec((block, F), lambda i: (i, _I0)),
            pl.BlockSpec((block, F), lambda i: (i + N_PAD // block, _I0)),
        ],
        out_specs=pl.BlockSpec((block, F), lambda i: (i, _I0)),
        out_shape=jax.ShapeDtypeStruct((N_PAD, F), jnp.float32),
        interpret=_INTERPRET,
    )(p, p)


def _dense_body(z_ref, z1_ref, z2_ref, agg_ref, deg_ref,
                w_ref, b_ref, t_ref, s1_ref, s2_ref):
    z = z_ref[...]
    xn = (jnp.dot(z, w_ref[0], preferred_element_type=jnp.float32)
          + deg_ref[...] * jnp.dot(z, w_ref[1], preferred_element_type=jnp.float32)
          + jnp.dot(z1_ref[...], w_ref[2], preferred_element_type=jnp.float32)
          + jnp.dot(z2_ref[...], w_ref[3], preferred_element_type=jnp.float32)
          + jnp.dot(agg_ref[...], w_ref[4], preferred_element_type=jnp.float32)
          + b_ref[...])
    t = xn[:, :F] + jax.nn.relu(xn[:, F:])
    t_ref[...] = t
    s1_ref[...] = jnp.sum(t, axis=0, keepdims=True)[None]
    s2_ref[...] = jnp.sum(t * t, axis=0, keepdims=True)[None]


def _dense_stage(z, z1, z2, agg, deg, w_stack, b_sum, block):
    m = z.shape[0]
    grid = m // block
    t, s1, s2 = pl.pallas_call(
        _dense_body,
        grid=(grid,),
        in_specs=[
            pl.BlockSpec((block, F), lambda i: (i, _I0)),
            pl.BlockSpec((block, F), lambda i: (i, _I0)),
            pl.BlockSpec((block, F), lambda i: (i, _I0)),
            pl.BlockSpec((block, F), lambda i: (i, _I0)),
            pl.BlockSpec((block, 1), lambda i: (i, _I0)),
            pl.BlockSpec((5, F, TWO_F), lambda i: (_I0, _I0, _I0)),
            pl.BlockSpec((1, TWO_F), lambda i: (_I0, _I0)),
        ],
        out_specs=[
            pl.BlockSpec((block, F), lambda i: (i, _I0)),
            pl.BlockSpec((1, 1, F), lambda i: (i, _I0, _I0)),
            pl.BlockSpec((1, 1, F), lambda i: (i, _I0, _I0)),
        ],
        out_shape=[
            jax.ShapeDtypeStruct((m, F), jnp.float32),
            jax.ShapeDtypeStruct((grid, 1, F), jnp.float32),
            jax.ShapeDtypeStruct((grid, 1, F), jnp.float32),
        ],
        interpret=_INTERPRET,
    )(z, z1, z2, agg, deg, w_stack, b_sum)
    return t, s1, s2


def _norm_body(t_ref, a_ref, c_ref, o_ref):
    o_ref[...] = t_ref[...] * a_ref[...] + c_ref[...]


def _norm_stage(t, a, c, block):
    m = t.shape[0]
    return pl.pallas_call(
        _norm_body,
        grid=(m // block,),
        in_specs=[
            pl.BlockSpec((block, F), lambda i: (i, _I0)),
            pl.BlockSpec((1, F), lambda i: (_I0, _I0)),
            pl.BlockSpec((1, F), lambda i: (_I0, _I0)),
        ],
        out_specs=pl.BlockSpec((block, F), lambda i: (i, _I0)),
        out_shape=jax.ShapeDtypeStruct((m, F), jnp.float32),
        interpret=_INTERPRET,
    )(t, a, c)


def _branch(z, z1, z2, agg, deg, names, params, bn_w, bn_b, block):
    w_stack = jnp.stack([params[n][0] for n in names])
    b_sum = sum(params[n][1] for n in names).reshape(1, TWO_F).astype(jnp.float32)
    t, s1, s2 = _dense_stage(z, z1, z2, agg, deg, w_stack, b_sum, block)
    m = jnp.float32(z.shape[0])
    mean = jnp.sum(s1, axis=(0, 1)) / m
    var = jnp.sum(s2, axis=(0, 1)) / m - mean * mean
    rstd = lax.rsqrt(var + 1e-5)
    a = (rstd * bn_w).reshape(1, F)
    c = (bn_b - mean * rstd * bn_w).reshape(1, F)
    return _norm_stage(t, a, c, block)


def _pad_edges_g(src, dst):
    npad = EP_G - E
    pad_src = (jnp.arange(npad, dtype=jnp.int32) % N)
    pad_dst = N + (jnp.arange(npad, dtype=jnp.int32) % (N_PAD - N))
    return (jnp.concatenate([src, pad_src]), jnp.concatenate([dst, pad_dst]))


def kernel(x, y, deg_g, deg_lg, params, edge_index, edge_index_lg, eid2nid):
    src_g = edge_index[0].astype(jnp.int32)
    dst_g = edge_index[1].astype(jnp.int32)
    src_l = edge_index_lg[0].astype(jnp.int32)
    dst_l = edge_index_lg[1].astype(jnp.int32)
    nid = eid2nid.astype(jnp.int32)

    src_gp, dst_gp = _pad_edges_g(src_g, dst_g)
    pos_p, _ = _pad_edges_g(jnp.arange(E, dtype=jnp.int32), dst_g)

    z1g_p = _segsum_g(x, src_gp, dst_gp)
    z1g = _combine(z1g_p)
    z2g_p = _segsum_g(z1g, src_gp, dst_gp)
    z2g = _combine(z2g_p)
    yx_p = _segsum_g(y, pos_p, dst_gp)
    yx = _combine(yx_p)
    x_out = _branch(x, z1g, z2g, yx, deg_g,
                    ["theta_x", "theta_deg", "theta_0", "theta_1", "theta_y"],
                    params, params["bn_x_w"], params["bn_x_b"], 1000)

    xy = jnp.take(x, nid, axis=0)
    z1l = jax.ops.segment_sum(y[src_l], dst_l, num_segments=E)
    z2l = jax.ops.segment_sum(z1l[src_l], dst_l, num_segments=E)
    xy_agg = jax.ops.segment_sum(xy[src_l], dst_l, num_segments=E)
    y_out = _branch(y, z1l, z2l, xy_agg, deg_lg,
                    ["gamma_y", "gamma_deg", "gamma_0", "gamma_1", "gamma_x"],
                    params, params["bn_y_w"], params["bn_y_b"], 1000)
    return (x_out, y_out)


# speedup vs baseline: 1.6677x; 1.0156x over previous
import functools

import jax
import jax.numpy as jnp
import numpy as _np
from jax import lax
from jax.experimental import pallas as pl
from jax.experimental.pallas import tpu as pltpu
from jax.experimental.pallas import tpu_sc as plsc

N = 10000
E = 320000
E_LG = 640000
F = 128
TWO_F = 256
NW = 32
T = 128

_INTERPRET = False
_I0 = _np.int32(0)

N_PAD = 10240
NTW_G = 79
EP_G = NW * NTW_G * T
ROWS_PER_SUB = N_PAD // 16
ZB = 64


def _make_segsum_g():
    mesh = plsc.VectorSubcoreMesh(core_axis_name="c", subcore_axis_name="s")

    @functools.partial(
        pl.kernel,
        mesh=mesh,
        out_type=jax.ShapeDtypeStruct((2 * N_PAD, F), jnp.float32),
        scratch_types=[
            pltpu.VMEM((2, T), jnp.int32),
            pltpu.VMEM((2, T), jnp.int32),
            pltpu.VMEM((2, T, F), jnp.float32),
            pltpu.VMEM((ZB, F), jnp.float32),
            pltpu.VMEM_SHARED((N_PAD, F), jnp.float32),
            pltpu.SemaphoreType.DMA,
            pltpu.SemaphoreType.DMA,
            pltpu.SemaphoreType.DMA,
        ],
    )
    def seg_kernel(vals_hbm, src_hbm, dst_hbm, out_hbm,
                   sidx, sdst, grows, zbuf, accs, sem, sem_i, sem_g):
        sc = lax.axis_index("c")
        sub = lax.axis_index("s")
        w = sc * 16 + sub
        zero16f = jnp.zeros((16,), jnp.float32)
        for r in range(ZB):
            for k in range(F // 16):
                zbuf[r, pl.ds(k * 16, 16)] = zero16f
        row0 = sub * ROWS_PER_SUB
        for j in range(ROWS_PER_SUB // ZB):
            pltpu.async_copy(zbuf, accs.at[pl.ds(row0 + j * ZB, ZB)],
                             sem).wait()
        plsc.subcore_barrier()

        base = w * (NTW_G * T)

        def fire_idx(j, b):
            pos = base + j * T
            d1 = pltpu.async_copy(src_hbm.at[pl.ds(pos, T)],
                                  sidx.at[jnp.int32(b)], sem_i)
            d2 = pltpu.async_copy(dst_hbm.at[pl.ds(pos, T)],
                                  sdst.at[jnp.int32(b)], sem_i)
            return (d1, d2)

        def fire_gather(b):
            return pltpu.async_copy(vals_hbm.at[sidx.at[jnp.int32(b)]],
                                    grows.at[jnp.int32(b)], sem_g)

        di = fire_idx(0, 0)
        di[0].wait()
        di[1].wait()
        dg = fire_gather(0)
        for j in range(NTW_G):
            b = j % 2
            if j + 1 < NTW_G:
                di = fire_idx(j + 1, 1 - b)
            dg.wait()
            if j + 1 < NTW_G:
                di[0].wait()
                di[1].wait()
                dg = fire_gather(1 - b)
            pltpu.async_copy(grows.at[jnp.int32(b)],
                             accs.at[sdst.at[jnp.int32(b)]], sem,
                             add=True).wait()
        plsc.subcore_barrier()
        pltpu.async_copy(
            accs.at[pl.ds(row0, ROWS_PER_SUB)],
            out_hbm.at[pl.ds(sc * N_PAD + row0, ROWS_PER_SUB)], sem).wait()

    return seg_kernel


_SEG_G = _make_segsum_g()


def _segsum_g(vals, src_pad, dst_pad):
    return _SEG_G(vals, src_pad, dst_pad)


def _add_body(a_ref, b_ref, o_ref):
    o_ref[...] = a_ref[...] + b_ref[...]


def _combine(p, block=1024):
    return pl.pallas_call(
        _add_body,
        grid=(N_PAD // block,),
        in_specs=[
            pl.BlockSpec((block, F), lambda i: (i, _I0)),
            pl.BlockSpec((block, F), lambda i: (i + N_PAD // block, _I0)),
        ],
        out_specs=pl.BlockSpec((block, F), lambda i: (i, _I0)),
        out_shape=jax.ShapeDtypeStruct((N_PAD, F), jnp.float32),
        interpret=_INTERPRET,
    )(p, p)


def _dense_body(z_ref, z1_ref, z2_ref, agg_ref, deg_ref,
                w_ref, b_ref, t_ref, s1_ref, s2_ref):
    z = z_ref[...]
    xn = (jnp.dot(z, w_ref[0], preferred_element_type=jnp.float32)
          + deg_ref[...] * jnp.dot(z, w_ref[1], preferred_element_type=jnp.float32)
          + jnp.dot(z1_ref[...], w_ref[2], preferred_element_type=jnp.float32)
          + jnp.dot(z2_ref[...], w_ref[3], preferred_element_type=jnp.float32)
          + jnp.dot(agg_ref[...], w_ref[4], preferred_element_type=jnp.float32)
          + b_ref[...])
    t = xn[:, :F] + jax.nn.relu(xn[:, F:])
    t_ref[...] = t
    s1_ref[...] = jnp.sum(t, axis=0, keepdims=True)[None]
    s2_ref[...] = jnp.sum(t * t, axis=0, keepdims=True)[None]


def _dense_stage(z, z1, z2, agg, deg, w_stack, b_sum, block):
    m = z.shape[0]
    grid = m // block
    t, s1, s2 = pl.pallas_call(
        _dense_body,
        grid=(grid,),
        in_specs=[
            pl.BlockSpec((block, F), lambda i: (i, _I0)),
            pl.BlockSpec((block, F), lambda i: (i, _I0)),
            pl.BlockSpec((block, F), lambda i: (i, _I0)),
            pl.BlockSpec((block, F), lambda i: (i, _I0)),
            pl.BlockSpec((block, 1), lambda i: (i, _I0)),
            pl.BlockSpec((5, F, TWO_F), lambda i: (_I0, _I0, _I0)),
            pl.BlockSpec((1, TWO_F), lambda i: (_I0, _I0)),
        ],
        out_specs=[
            pl.BlockSpec((block, F), lambda i: (i, _I0)),
            pl.BlockSpec((1, 1, F), lambda i: (i, _I0, _I0)),
            pl.BlockSpec((1, 1, F), lambda i: (i, _I0, _I0)),
        ],
        out_shape=[
            jax.ShapeDtypeStruct((m, F), jnp.float32),
            jax.ShapeDtypeStruct((grid, 1, F), jnp.float32),
            jax.ShapeDtypeStruct((grid, 1, F), jnp.float32),
        ],
        interpret=_INTERPRET,
    )(z, z1, z2, agg, deg, w_stack, b_sum)
    return t, s1, s2


def _norm_body(t_ref, a_ref, c_ref, o_ref):
    o_ref[...] = t_ref[...] * a_ref[...] + c_ref[...]


def _norm_stage(t, a, c, block):
    m = t.shape[0]
    return pl.pallas_call(
        _norm_body,
        grid=(m // block,),
        in_specs=[
            pl.BlockSpec((block, F), lambda i: (i, _I0)),
            pl.BlockSpec((1, F), lambda i: (_I0, _I0)),
            pl.BlockSpec((1, F), lambda i: (_I0, _I0)),
        ],
        out_specs=pl.BlockSpec((block, F), lambda i: (i, _I0)),
        out_shape=jax.ShapeDtypeStruct((m, F), jnp.float32),
        interpret=_INTERPRET,
    )(t, a, c)


def _branch(z, z1, z2, agg, deg, names, params, bn_w, bn_b, block):
    w_stack = jnp.stack([params[n][0] for n in names])
    b_sum = sum(params[n][1] for n in names).reshape(1, TWO_F).astype(jnp.float32)
    t, s1, s2 = _dense_stage(z, z1, z2, agg, deg, w_stack, b_sum, block)
    m = jnp.float32(z.shape[0])
    mean = jnp.sum(s1, axis=(0, 1)) / m
    var = jnp.sum(s2, axis=(0, 1)) / m - mean * mean
    rstd = lax.rsqrt(var + 1e-5)
    a = (rstd * bn_w).reshape(1, F)
    c = (bn_b - mean * rstd * bn_w).reshape(1, F)
    return _norm_stage(t, a, c, block)


def _pad_edges_g(src, dst):
    npad = EP_G - E
    pad_src = (jnp.arange(npad, dtype=jnp.int32) % N)
    pad_dst = N + (jnp.arange(npad, dtype=jnp.int32) % (N_PAD - N))
    return (jnp.concatenate([src, pad_src]), jnp.concatenate([dst, pad_dst]))


def kernel(x, y, deg_g, deg_lg, params, edge_index, edge_index_lg, eid2nid):
    src_g = edge_index[0].astype(jnp.int32)
    dst_g = edge_index[1].astype(jnp.int32)
    src_l = edge_index_lg[0].astype(jnp.int32)
    dst_l = edge_index_lg[1].astype(jnp.int32)
    nid = eid2nid.astype(jnp.int32)

    src_gp, dst_gp = _pad_edges_g(src_g, dst_g)
    pos_p, _ = _pad_edges_g(jnp.arange(E, dtype=jnp.int32), dst_g)

    z1g_p = _segsum_g(x, src_gp, dst_gp)
    z1g = _combine(z1g_p)
    z2g_p = _segsum_g(z1g, src_gp, dst_gp)
    z2g = _combine(z2g_p)
    yx_p = _segsum_g(y, pos_p, dst_gp)
    yx = _combine(yx_p)
    x_out = _branch(x, z1g, z2g, yx, deg_g,
                    ["theta_x", "theta_deg", "theta_0", "theta_1", "theta_y"],
                    params, params["bn_x_w"], params["bn_x_b"], 1000)

    xy = jnp.take(x, nid, axis=0)
    z1l = jax.ops.segment_sum(y[src_l], dst_l, num_segments=E)
    z2l = jax.ops.segment_sum(z1l[src_l], dst_l, num_segments=E)
    xy_agg = jax.ops.segment_sum(xy[src_l], dst_l, num_segments=E)
    y_out = _branch(y, z1l, z2l, xy_agg, deg_lg,
                    ["gamma_y", "gamma_deg", "gamma_0", "gamma_1", "gamma_x"],
                    params, params["bn_y_w"], params["bn_y_b"], 1000)
    return (x_out, y_out)


# speedup vs baseline: 1.6994x; 1.0190x over previous
import functools

import jax
import jax.numpy as jnp
import numpy as _np
from jax import lax
from jax.experimental import pallas as pl
from jax.experimental.pallas import tpu as pltpu
from jax.experimental.pallas import tpu_sc as plsc

N = 10000
E = 320000
E_LG = 640000
F = 128
TWO_F = 256
NW = 32
T = 128

_INTERPRET = False
_I0 = _np.int32(0)

N_PAD = 10240
NTW_G = 79
EP_G = NW * NTW_G * T
ROWS_PER_SUB = N_PAD // 16
ZB = 64


def _make_segsum_g():
    mesh = plsc.VectorSubcoreMesh(core_axis_name="c", subcore_axis_name="s")

    @functools.partial(
        pl.kernel,
        mesh=mesh,
        out_type=jax.ShapeDtypeStruct((2 * N_PAD, F), jnp.float32),
        scratch_types=[
            pltpu.VMEM((2, T), jnp.int32),
            pltpu.VMEM((2, T), jnp.int32),
            pltpu.VMEM((2, T, F), jnp.float32),
            pltpu.VMEM((ZB, F), jnp.float32),
            pltpu.VMEM_SHARED((N_PAD, F), jnp.float32),
            pltpu.SemaphoreType.DMA,
            pltpu.SemaphoreType.DMA,
            pltpu.SemaphoreType.DMA,
        ],
    )
    def seg_kernel(vals_hbm, src_hbm, dst_hbm, out_hbm,
                   sidx, sdst, grows, zbuf, accs, sem, sem_i, sem_g):
        sc = lax.axis_index("c")
        sub = lax.axis_index("s")
        w = sc * 16 + sub
        zero16f = jnp.zeros((16,), jnp.float32)
        for r in range(ZB):
            for k in range(F // 16):
                zbuf[r, pl.ds(k * 16, 16)] = zero16f
        row0 = sub * ROWS_PER_SUB
        for j in range(ROWS_PER_SUB // ZB):
            pltpu.async_copy(zbuf, accs.at[pl.ds(row0 + j * ZB, ZB)],
                             sem).wait()
        plsc.subcore_barrier()

        base = w * (NTW_G * T)

        def fire_idx(j, b):
            pos = base + j * T
            d1 = pltpu.async_copy(src_hbm.at[pl.ds(pos, T)],
                                  sidx.at[jnp.int32(b)], sem_i)
            d2 = pltpu.async_copy(dst_hbm.at[pl.ds(pos, T)],
                                  sdst.at[jnp.int32(b)], sem_i)
            return (d1, d2)

        def fire_gather(b):
            return pltpu.async_copy(vals_hbm.at[sidx.at[jnp.int32(b)]],
                                    grows.at[jnp.int32(b)], sem_g)

        di = fire_idx(0, 0)
        di[0].wait()
        di[1].wait()
        dg = fire_gather(0)
        for j in range(NTW_G):
            b = j % 2
            if j + 1 < NTW_G:
                di = fire_idx(j + 1, 1 - b)
            dg.wait()
            if j + 1 < NTW_G:
                di[0].wait()
                di[1].wait()
                dg = fire_gather(1 - b)
            pltpu.async_copy(grows.at[jnp.int32(b)],
                             accs.at[sdst.at[jnp.int32(b)]], sem,
                             add=True).wait()
        plsc.subcore_barrier()
        pltpu.async_copy(
            accs.at[pl.ds(row0, ROWS_PER_SUB)],
            out_hbm.at[pl.ds(sc * N_PAD + row0, ROWS_PER_SUB)], sem).wait()

    return seg_kernel


_SEG_G = _make_segsum_g()


def _segsum_g(vals, src_pad, dst_pad):
    return _SEG_G(vals, src_pad, dst_pad)


def _add_body(a_ref, b_ref, o_ref):
    o_ref[...] = a_ref[...] + b_ref[...]


def _combine(p, block=1024):
    return pl.pallas_call(
        _add_body,
        grid=(N_PAD // block,),
        in_specs=[
            pl.BlockSpec((block, F), lambda i: (i, _I0)),
            pl.BlockSpec((block, F), lambda i: (i + N_PAD // block, _I0)),
        ],
        out_specs=pl.BlockSpec((block, F), lambda i: (i, _I0)),
        out_shape=jax.ShapeDtypeStruct((N_PAD, F), jnp.float32),
        interpret=_INTERPRET,
    )(p, p)


def _dense_body(z_ref, z1_ref, z2_ref, agg_ref, deg_ref,
                w_ref, b_ref, t_ref, s1_ref, s2_ref):
    z = z_ref[...]
    xn = (jnp.dot(z, w_ref[0], preferred_element_type=jnp.float32)
          + deg_ref[...] * jnp.dot(z, w_ref[1], preferred_element_type=jnp.float32)
          + jnp.dot(z1_ref[...], w_ref[2], preferred_element_type=jnp.float32)
          + jnp.dot(z2_ref[...], w_ref[3], preferred_element_type=jnp.float32)
          + jnp.dot(agg_ref[...], w_ref[4], preferred_element_type=jnp.float32)
          + b_ref[...])
    t = xn[:, :F] + jax.nn.relu(xn[:, F:])
    t_ref[...] = t
    s1_ref[...] = jnp.sum(t, axis=0, keepdims=True)[None]
    s2_ref[...] = jnp.sum(t * t, axis=0, keepdims=True)[None]


def _dense_stage(z, z1, z2, agg, deg, w_stack, b_sum, block):
    m = z.shape[0]
    grid = m // block
    t, s1, s2 = pl.pallas_call(
        _dense_body,
        grid=(grid,),
        in_specs=[
            pl.BlockSpec((block, F), lambda i: (i, _I0)),
            pl.BlockSpec((block, F), lambda i: (i, _I0)),
            pl.BlockSpec((block, F), lambda i: (i, _I0)),
            pl.BlockSpec((block, F), lambda i: (i, _I0)),
            pl.BlockSpec((block, 1), lambda i: (i, _I0)),
            pl.BlockSpec((5, F, TWO_F), lambda i: (_I0, _I0, _I0)),
            pl.BlockSpec((1, TWO_F), lambda i: (_I0, _I0)),
        ],
        out_specs=[
            pl.BlockSpec((block, F), lambda i: (i, _I0)),
            pl.BlockSpec((1, 1, F), lambda i: (i, _I0, _I0)),
            pl.BlockSpec((1, 1, F), lambda i: (i, _I0, _I0)),
        ],
        out_shape=[
            jax.ShapeDtypeStruct((m, F), jnp.float32),
            jax.ShapeDtypeStruct((grid, 1, F), jnp.float32),
            jax.ShapeDtypeStruct((grid, 1, F), jnp.float32),
        ],
        interpret=_INTERPRET,
    )(z, z1, z2, agg, deg, w_stack, b_sum)
    return t, s1, s2


def _norm_body(t_ref, a_ref, c_ref, o_ref):
    o_ref[...] = t_ref[...] * a_ref[...] + c_ref[...]


def _norm_stage(t, a, c, block):
    m = t.shape[0]
    return pl.pallas_call(
        _norm_body,
        grid=(m // block,),
        in_specs=[
            pl.BlockSpec((block, F), lambda i: (i, _I0)),
            pl.BlockSpec((1, F), lambda i: (_I0, _I0)),
            pl.BlockSpec((1, F), lambda i: (_I0, _I0)),
        ],
        out_specs=pl.BlockSpec((block, F), lambda i: (i, _I0)),
        out_shape=jax.ShapeDtypeStruct((m, F), jnp.float32),
        interpret=_INTERPRET,
    )(t, a, c)


def _branch(z, z1, z2, agg, deg, names, params, bn_w, bn_b, block):
    w_stack = jnp.stack([params[n][0] for n in names])
    b_sum = sum(params[n][1] for n in names).reshape(1, TWO_F).astype(jnp.float32)
    t, s1, s2 = _dense_stage(z, z1, z2, agg, deg, w_stack, b_sum, block)
    m = jnp.float32(z.shape[0])
    mean = jnp.sum(s1, axis=(0, 1)) / m
    var = jnp.sum(s2, axis=(0, 1)) / m - mean * mean
    rstd = lax.rsqrt(var + 1e-5)
    a = (rstd * bn_w).reshape(1, F)
    c = (bn_b - mean * rstd * bn_w).reshape(1, F)
    return _norm_stage(t, a, c, block)


def _pad_edges_g(src, dst):
    npad = EP_G - E
    pad_src = (jnp.arange(npad, dtype=jnp.int32) % N)
    pad_dst = N + (jnp.arange(npad, dtype=jnp.int32) % (N_PAD - N))
    return (jnp.concatenate([src, pad_src]), jnp.concatenate([dst, pad_dst]))


def kernel(x, y, deg_g, deg_lg, params, edge_index, edge_index_lg, eid2nid):
    src_g = edge_index[0].astype(jnp.int32)
    dst_g = edge_index[1].astype(jnp.int32)
    src_l = edge_index_lg[0].astype(jnp.int32)
    dst_l = edge_index_lg[1].astype(jnp.int32)
    nid = eid2nid.astype(jnp.int32)

    src_gp, dst_gp = _pad_edges_g(src_g, dst_g)
    pos_p, _ = _pad_edges_g(jnp.arange(E, dtype=jnp.int32), dst_g)

    z1g_p = _segsum_g(x, src_gp, dst_gp)
    z1g = _combine(z1g_p)
    z2g_p = _segsum_g(z1g, src_gp, dst_gp)
    z2g = _combine(z2g_p)
    yx_p = _segsum_g(y, pos_p, dst_gp)
    yx = _combine(yx_p)
    x_out = _branch(x, z1g, z2g, yx, deg_g,
                    ["theta_x", "theta_deg", "theta_0", "theta_1", "theta_y"],
                    params, params["bn_x_w"], params["bn_x_b"], 2000)

    xy = jnp.take(x, nid, axis=0)
    z1l = jax.ops.segment_sum(y[src_l], dst_l, num_segments=E)
    z2l = jax.ops.segment_sum(z1l[src_l], dst_l, num_segments=E)
    xy_agg = jax.ops.segment_sum(xy[src_l], dst_l, num_segments=E)
    y_out = _branch(y, z1l, z2l, xy_agg, deg_lg,
                    ["gamma_y", "gamma_deg", "gamma_0", "gamma_1", "gamma_x"],
                    params, params["bn_y_w"], params["bn_y_b"], 2000)
    return (x_out, y_out)


# speedup vs baseline: 1.7157x; 1.0096x over previous
import functools

import jax
import jax.numpy as jnp
import numpy as _np
from jax import lax
from jax.experimental import pallas as pl
from jax.experimental.pallas import tpu as pltpu
from jax.experimental.pallas import tpu_sc as plsc

N = 10000
E = 320000
E_LG = 640000
F = 128
TWO_F = 256
NW = 32
T = 128

_INTERPRET = False
_I0 = _np.int32(0)

N_PAD = 10240
NTW_G = 79
EP_G = NW * NTW_G * T
ROWS_PER_SUB = N_PAD // 16
ZB = 64


def _make_segsum_g():
    mesh = plsc.VectorSubcoreMesh(core_axis_name="c", subcore_axis_name="s")

    @functools.partial(
        pl.kernel,
        mesh=mesh,
        out_type=jax.ShapeDtypeStruct((2 * N_PAD, F), jnp.float32),
        scratch_types=[
            pltpu.VMEM((2, T), jnp.int32),
            pltpu.VMEM((2, T), jnp.int32),
            pltpu.VMEM((2, T, F), jnp.float32),
            pltpu.VMEM((ZB, F), jnp.float32),
            pltpu.VMEM_SHARED((N_PAD, F), jnp.float32),
            pltpu.SemaphoreType.DMA,
            pltpu.SemaphoreType.DMA,
            pltpu.SemaphoreType.DMA,
        ],
    )
    def seg_kernel(vals_hbm, src_hbm, dst_hbm, out_hbm,
                   sidx, sdst, grows, zbuf, accs, sem, sem_i, sem_g):
        sc = lax.axis_index("c")
        sub = lax.axis_index("s")
        w = sc * 16 + sub
        zero16f = jnp.zeros((16,), jnp.float32)
        for r in range(ZB):
            for k in range(F // 16):
                zbuf[r, pl.ds(k * 16, 16)] = zero16f
        row0 = sub * ROWS_PER_SUB
        for j in range(ROWS_PER_SUB // ZB):
            pltpu.async_copy(zbuf, accs.at[pl.ds(row0 + j * ZB, ZB)],
                             sem).wait()
        plsc.subcore_barrier()

        base = w * (NTW_G * T)

        def fire_idx(j, b):
            pos = base + j * T
            d1 = pltpu.async_copy(src_hbm.at[pl.ds(pos, T)],
                                  sidx.at[jnp.int32(b)], sem_i)
            d2 = pltpu.async_copy(dst_hbm.at[pl.ds(pos, T)],
                                  sdst.at[jnp.int32(b)], sem_i)
            return (d1, d2)

        def fire_gather(b):
            return pltpu.async_copy(vals_hbm.at[sidx.at[jnp.int32(b)]],
                                    grows.at[jnp.int32(b)], sem_g)

        di = fire_idx(0, 0)
        di[0].wait()
        di[1].wait()
        dg = fire_gather(0)
        for j in range(NTW_G):
            b = j % 2
            if j + 1 < NTW_G:
                di = fire_idx(j + 1, 1 - b)
            dg.wait()
            if j + 1 < NTW_G:
                di[0].wait()
                di[1].wait()
                dg = fire_gather(1 - b)
            pltpu.async_copy(grows.at[jnp.int32(b)],
                             accs.at[sdst.at[jnp.int32(b)]], sem,
                             add=True).wait()
        plsc.subcore_barrier()
        pltpu.async_copy(
            accs.at[pl.ds(row0, ROWS_PER_SUB)],
            out_hbm.at[pl.ds(sc * N_PAD + row0, ROWS_PER_SUB)], sem).wait()

    return seg_kernel


_SEG_G = _make_segsum_g()


def _segsum_g(vals, src_pad, dst_pad):
    return _SEG_G(vals, src_pad, dst_pad)


def _add_body(a_ref, b_ref, o_ref):
    o_ref[...] = a_ref[...] + b_ref[...]


def _combine(p, block=1024):
    return pl.pallas_call(
        _add_body,
        grid=(N_PAD // block,),
        in_specs=[
            pl.BlockSpec((block, F), lambda i: (i, _I0)),
            pl.BlockSpec((block, F), lambda i: (i + N_PAD // block, _I0)),
        ],
        out_specs=pl.BlockSpec((block, F), lambda i: (i, _I0)),
        out_shape=jax.ShapeDtypeStruct((N_PAD, F), jnp.float32),
        interpret=_INTERPRET,
    )(p, p)


def _dense_body(z_ref, z1_ref, z2_ref, agg_ref, deg_ref,
                w_ref, b_ref, t_ref, s1_ref, s2_ref):
    z = z_ref[...]
    xn = (jnp.dot(z, w_ref[0], preferred_element_type=jnp.float32)
          + deg_ref[...] * jnp.dot(z, w_ref[1], preferred_element_type=jnp.float32)
          + jnp.dot(z1_ref[...], w_ref[2], preferred_element_type=jnp.float32)
          + jnp.dot(z2_ref[...], w_ref[3], preferred_element_type=jnp.float32)
          + jnp.dot(agg_ref[...], w_ref[4], preferred_element_type=jnp.float32)
          + b_ref[...])
    t = xn[:, :F] + jax.nn.relu(xn[:, F:])
    t_ref[...] = t
    s1_ref[...] = jnp.sum(t, axis=0, keepdims=True)[None]
    s2_ref[...] = jnp.sum(t * t, axis=0, keepdims=True)[None]


def _dense_stage(z, z1, z2, agg, deg, w_stack, b_sum, block):
    m = z.shape[0]
    grid = m // block
    t, s1, s2 = pl.pallas_call(
        _dense_body,
        grid=(grid,),
        in_specs=[
            pl.BlockSpec((block, F), lambda i: (i, _I0)),
            pl.BlockSpec((block, F), lambda i: (i, _I0)),
            pl.BlockSpec((block, F), lambda i: (i, _I0)),
            pl.BlockSpec((block, F), lambda i: (i, _I0)),
            pl.BlockSpec((block, 1), lambda i: (i, _I0)),
            pl.BlockSpec((5, F, TWO_F), lambda i: (_I0, _I0, _I0)),
            pl.BlockSpec((1, TWO_F), lambda i: (_I0, _I0)),
        ],
        out_specs=[
            pl.BlockSpec((block, F), lambda i: (i, _I0)),
            pl.BlockSpec((1, 1, F), lambda i: (i, _I0, _I0)),
            pl.BlockSpec((1, 1, F), lambda i: (i, _I0, _I0)),
        ],
        out_shape=[
            jax.ShapeDtypeStruct((m, F), jnp.float32),
            jax.ShapeDtypeStruct((grid, 1, F), jnp.float32),
            jax.ShapeDtypeStruct((grid, 1, F), jnp.float32),
        ],
        interpret=_INTERPRET,
    )(z, z1, z2, agg, deg, w_stack, b_sum)
    return t, s1, s2


def _norm_body(t_ref, a_ref, c_ref, o_ref):
    o_ref[...] = t_ref[...] * a_ref[...] + c_ref[...]


def _norm_stage(t, a, c, block):
    m = t.shape[0]
    return pl.pallas_call(
        _norm_body,
        grid=(m // block,),
        in_specs=[
            pl.BlockSpec((block, F), lambda i: (i, _I0)),
            pl.BlockSpec((1, F), lambda i: (_I0, _I0)),
            pl.BlockSpec((1, F), lambda i: (_I0, _I0)),
        ],
        out_specs=pl.BlockSpec((block, F), lambda i: (i, _I0)),
        out_shape=jax.ShapeDtypeStruct((m, F), jnp.float32),
        interpret=_INTERPRET,
    )(t, a, c)


def _branch(z, z1, z2, agg, deg, names, params, bn_w, bn_b, block):
    w_stack = jnp.stack([params[n][0] for n in names])
    b_sum = sum(params[n][1] for n in names).reshape(1, TWO_F).astype(jnp.float32)
    t, s1, s2 = _dense_stage(z, z1, z2, agg, deg, w_stack, b_sum, block)
    m = jnp.float32(z.shape[0])
    mean = jnp.sum(s1, axis=(0, 1)) / m
    var = jnp.sum(s2, axis=(0, 1)) / m - mean * mean
    rstd = lax.rsqrt(var + 1e-5)
    a = (rstd * bn_w).reshape(1, F)
    c = (bn_b - mean * rstd * bn_w).reshape(1, F)
    return _norm_stage(t, a, c, block)


def _pad_edges_g(src, dst):
    npad = EP_G - E
    pad_src = (jnp.arange(npad, dtype=jnp.int32) % N)
    pad_dst = N + (jnp.arange(npad, dtype=jnp.int32) % (N_PAD - N))
    return (jnp.concatenate([src, pad_src]), jnp.concatenate([dst, pad_dst]))


def kernel(x, y, deg_g, deg_lg, params, edge_index, edge_index_lg, eid2nid):
    src_g = edge_index[0].astype(jnp.int32)
    dst_g = edge_index[1].astype(jnp.int32)
    src_l = edge_index_lg[0].astype(jnp.int32)
    dst_l = edge_index_lg[1].astype(jnp.int32)
    nid = eid2nid.astype(jnp.int32)

    src_gp, dst_gp = _pad_edges_g(src_g, dst_g)
    pos_p, _ = _pad_edges_g(jnp.arange(E, dtype=jnp.int32), dst_g)

    z1g_p = _segsum_g(x, src_gp, dst_gp)
    z1g = _combine(z1g_p)
    z2g_p = _segsum_g(z1g, src_gp, dst_gp)
    z2g = _combine(z2g_p)
    yx_p = _segsum_g(y, pos_p, dst_gp)
    yx = _combine(yx_p)
    x_out = _branch(x, z1g, z2g, yx, deg_g,
                    ["theta_x", "theta_deg", "theta_0", "theta_1", "theta_y"],
                    params, params["bn_x_w"], params["bn_x_b"], 5000)

    xy = jnp.take(x, nid, axis=0)
    z1l = jax.ops.segment_sum(y[src_l], dst_l, num_segments=E)
    z2l = jax.ops.segment_sum(z1l[src_l], dst_l, num_segments=E)
    xy_agg = jax.ops.segment_sum(xy[src_l], dst_l, num_segments=E)
    y_out = _branch(y, z1l, z2l, xy_agg, deg_lg,
                    ["gamma_y", "gamma_deg", "gamma_0", "gamma_1", "gamma_x"],
                    params, params["bn_y_w"], params["bn_y_b"], 4000)
    return (x_out, y_out)
